# Initial kernel scaffold; baseline (speedup 1.0000x reference)
import jax
import jax.numpy as jnp
from jax.experimental import pallas as pl


def kernel(nodes, history_uv, history_r, history_w, v2e_w, u2e_w, r2e_w, word_emb, Wc3, bc3, Wc5, bc5, W1, b1, W2, b2, A1, a1, A2, a2, A3, a3):
    raise NotImplementedError("write your pallas kernel here")


# speedup vs baseline: 2.0817x; 2.0817x over previous
import functools

import jax
import jax.numpy as jnp
from jax import lax
from jax.experimental import pallas as pl
from jax.experimental.pallas import tpu as pltpu
from jax.experimental.pallas import tpu_sc as plsc

B = 256
HIST = 20
SEQ = 20
E = 100
WD = 64
NF = 100
NSEQ = B * HIST
NTOK = NSEQ * SEQ
EP = 104

_NC = 2
_NS = 16
NW = _NC * _NS

W_CHUNK = 128
W_PER = NTOK // NW
W_NCH = W_PER // W_CHUNK
V_CHUNK = 80
V_NCH = 2
V_PER = V_CHUNK * V_NCH
U_PER = B // NW

NB = 32
GRID = B // NB
NSB = NB * HIST


def _sc_gather_body(w_idx, w_tab, v_idx, v_tab, u_idx, u_tab,
                    w_out, v_out, u_out,
                    widx_v, wbuf, vidx_v, vbuf, uidx_v, ubuf, sem):
  c = lax.axis_index("c")
  s = lax.axis_index("s")
  wid = s * _NC + c

  pltpu.sync_copy(w_idx.at[wid], widx_v)

  def w_body(j, carry):
    pltpu.async_copy(w_tab.at[widx_v.at[j]], wbuf, sem).wait()
    pltpu.sync_copy(wbuf, w_out.at[pl.ds(wid * W_PER + j * W_CHUNK, W_CHUNK)])
    return carry

  lax.fori_loop(0, W_NCH, w_body, 0)

  pltpu.sync_copy(v_idx.at[wid], vidx_v)
  for j in range(V_NCH):
    pltpu.async_copy(v_tab.at[vidx_v.at[j]], vbuf, sem).wait()
    pltpu.sync_copy(vbuf, v_out.at[pl.ds(wid * V_PER + j * V_CHUNK, V_CHUNK)])

  pltpu.sync_copy(u_idx.at[pl.ds(wid * U_PER, U_PER)], uidx_v)
  pltpu.async_copy(u_tab.at[uidx_v], ubuf, sem).wait()
  pltpu.sync_copy(ubuf, u_out.at[pl.ds(wid * U_PER, U_PER)])


@functools.partial(jax.jit, static_argnames=())
def _sc_gather(w_idx2d, word_emb, v_idx2d, v2e_w, nodes, u2e_w):
  mesh = plsc.VectorSubcoreMesh(core_axis_name="c", subcore_axis_name="s")
  fn = functools.partial(
      pl.kernel,
      mesh=mesh,
      compiler_params=pltpu.CompilerParams(use_tc_tiling_on_sc=False),
      out_type=[
          jax.ShapeDtypeStruct((NTOK, WD), jnp.float32),
          jax.ShapeDtypeStruct((NSEQ, EP), jnp.float32),
          jax.ShapeDtypeStruct((B, EP), jnp.float32),
      ],
      scratch_types=[
          pltpu.VMEM((W_NCH, W_CHUNK), jnp.int32),
          pltpu.VMEM((W_CHUNK, WD), jnp.float32),
          pltpu.VMEM((V_NCH, V_CHUNK), jnp.int32),
          pltpu.VMEM((V_CHUNK, EP), jnp.float32),
          pltpu.VMEM((U_PER,), jnp.int32),
          pltpu.VMEM((U_PER, EP), jnp.float32),
          pltpu.SemaphoreType.DMA,
      ],
  )(_sc_gather_body)
  return fn(w_idx2d, word_emb, v_idx2d, v2e_w, nodes, u2e_w)


def _tc_body(emb_ref, euv_ref, u_ref, r_ref, r2e_ref,
             w3_ref, bc3_ref, w5_ref, bc5_ref,
             w1a_ref, w1b_ref, w1c3_ref, w1c5_ref, b1_ref,
             w2_ref, b2_ref,
             a1o_ref, a1u_ref, a1b_ref, a2_ref, a2b_ref, a3_ref, a3b_ref,
             out_ref):
  f32 = jnp.float32
  dot = lambda a, b: lax.dot(a, b, preferred_element_type=f32)
  relu = lambda x: jnp.maximum(x, 0.0)

  emb = emb_ref[...]

  w3 = w3_ref[...]
  acc3 = dot(emb[:, 0:3 * WD], w3)
  for l in range(1, SEQ - 3 + 1):
    acc3 = jnp.maximum(acc3, dot(emb[:, l * WD:(l + 3) * WD], w3))
  e3 = relu(acc3 + bc3_ref[...])

  w5 = w5_ref[...]
  acc5 = dot(emb[:, 0:5 * WD], w5)
  for l in range(1, SEQ - 5 + 1):
    acc5 = jnp.maximum(acc5, dot(emb[:, l * WD:(l + 5) * WD], w5))
  e5 = relu(acc5 + bc5_ref[...])

  euv = euv_ref[...]
  r = r_ref[...]
  onehot = (lax.broadcasted_iota(jnp.int32, (NSB, 8), 1) == r).astype(f32)
  rmat = dot(r2e_ref[...], w1b_ref[...])
  h = (dot(euv, w1a_ref[...]) + dot(onehot, rmat)
       + dot(e3, w1c3_ref[...]) + dot(e5, w1c5_ref[...]) + b1_ref[...])
  h = relu(h)
  o = relu(dot(h, w2_ref[...]) + b2_ref[...])

  u = u_ref[...]
  ua = dot(u, a1u_ref[...])
  ua_b = jnp.broadcast_to(ua[:, None, :], (NB, HIST, E)).reshape(NSB, E)
  a = relu(dot(o, a1o_ref[...]) + ua_b + a1b_ref[...])
  a = relu(dot(a, a2_ref[...]) + a2b_ref[...])
  s = dot(a, a3_ref[...]) + a3b_ref[...]

  srs = s.reshape(NB, HIST)
  m = jnp.max(srs, axis=1, keepdims=True)
  ex = jnp.exp(srs - m)
  wgt = ex / jnp.sum(ex, axis=1, keepdims=True)

  o3 = o.reshape(NB, HIST, E)
  out_ref[...] = jnp.sum(o3 * wgt[:, :, None], axis=1)


def _full_spec(shape):
  nd = len(shape)
  return pl.BlockSpec(shape, lambda i, _n=nd: (0,) * _n)


def kernel(nodes, history_uv, history_r, history_w,
           v2e_w, u2e_w, r2e_w, word_emb,
           Wc3, bc3, Wc5, bc5,
           W1, b1, W2, b2,
           A1, a1, A2, a2, A3, a3):
  f32 = jnp.float32
  nodes = nodes.astype(jnp.int32)
  w_idx2d = history_w.astype(jnp.int32).reshape(NW, W_NCH, W_CHUNK)
  v_idx2d = history_uv.astype(jnp.int32).reshape(NW, V_NCH, V_CHUNK)

  pad4 = ((0, 0), (0, EP - E))
  w_rows, v_rows, u_rows = _sc_gather(
      w_idx2d, word_emb, v_idx2d, jnp.pad(v2e_w, pad4), nodes,
      jnp.pad(u2e_w, pad4))
  emb = w_rows.reshape(NSEQ, SEQ * WD)

  w3 = Wc3.reshape(NF, 3 * WD).T
  w5 = Wc5.reshape(NF, 5 * WD).T
  w1t = W1.T
  w1a = jnp.pad(w1t[:E], ((0, EP - E), (0, 0)))
  w1b = w1t[E:2 * E]
  w1c3 = w1t[2 * E:2 * E + NF]
  w1c5 = w1t[2 * E + NF:]
  r2e8 = jnp.zeros((8, E), f32).at[:6].set(r2e_w)
  a1t = A1.T
  a1o = a1t[:E]
  a1u = jnp.pad(a1t[E:], ((0, EP - E), (0, 0)))

  grid_spec = pl.GridSpec(
      grid=(GRID,),
      in_specs=[
          pl.BlockSpec((NSB, SEQ * WD), lambda i: (i, 0)),
          pl.BlockSpec((NSB, EP), lambda i: (i, 0)),
          pl.BlockSpec((NB, EP), lambda i: (i, 0)),
          pl.BlockSpec((NSB, 1), lambda i: (i, 0)),
          _full_spec((8, E)),
          _full_spec((3 * WD, NF)),
          _full_spec((1, NF)),
          _full_spec((5 * WD, NF)),
          _full_spec((1, NF)),
          _full_spec((EP, E)),
          _full_spec((E, E)),
          _full_spec((NF, E)),
          _full_spec((NF, E)),
          _full_spec((1, E)),
          _full_spec((E, E)),
          _full_spec((1, E)),
          _full_spec((E, E)),
          _full_spec((EP, E)),
          _full_spec((1, E)),
          _full_spec((E, E)),
          _full_spec((1, E)),
          _full_spec((E, 1)),
          _full_spec((1, 1)),
      ],
      out_specs=pl.BlockSpec((NB, E), lambda i: (i, 0)),
  )

  out = pl.pallas_call(
      _tc_body,
      grid_spec=grid_spec,
      out_shape=jax.ShapeDtypeStruct((B, E), f32),
  )(
      emb, v_rows, u_rows, history_r.astype(jnp.int32).reshape(NSEQ, 1), r2e8,
      w3, bc3.reshape(1, NF), w5, bc5.reshape(1, NF),
      w1a, w1b, w1c3, w1c5, b1.reshape(1, E),
      W2.T, b2.reshape(1, E),
      a1o, a1u, a1.reshape(1, E), A2.T, a2.reshape(1, E),
      A3.T, a3.reshape(1, 1),
  )
  return out


# speedup vs baseline: 4.1436x; 1.9905x over previous
import functools

import jax
import jax.numpy as jnp
from jax import lax
from jax.experimental import pallas as pl
from jax.experimental.pallas import tpu as pltpu
from jax.experimental.pallas import tpu_sc as plsc

B = 256
HIST = 20
SEQ = 20
E = 100
WD = 64
NF = 100
NV = 100000
NSEQ = B * HIST
NTOK = NSEQ * SEQ
EP = 128

_NC = 2
_NS = 16
NW = _NC * _NS

W_CHUNK = 128
W_PER = NTOK // NW
W_NCH = W_PER // W_CHUNK
V_CHUNK = 128
V_NCH = NSEQ // V_CHUNK
U_NCH = B // V_CHUNK

NB = 32
GRID = B // NB
NSB = NB * HIST

PAD_ROWS = 4000
PAD_GRID = NV // PAD_ROWS


def _sc_word_body(w_idx, w_tab, w_out, widx_v, wbuf, sem):
  c = lax.axis_index("c")
  s = lax.axis_index("s")
  wid = s * _NC + c
  pltpu.sync_copy(w_idx.at[wid], widx_v)

  def w_body(j, carry):
    pltpu.async_copy(w_tab.at[widx_v.at[j]], wbuf, sem).wait()
    pltpu.sync_copy(wbuf, w_out.at[pl.ds(wid * W_PER + j * W_CHUNK, W_CHUNK)])
    return carry

  lax.fori_loop(0, W_NCH, w_body, 0)


def _sc_word(w_idx3d, word_emb):
  mesh = plsc.VectorSubcoreMesh(core_axis_name="c", subcore_axis_name="s")
  fn = functools.partial(
      pl.kernel,
      mesh=mesh,
      compiler_params=pltpu.CompilerParams(use_tc_tiling_on_sc=False),
      out_type=jax.ShapeDtypeStruct((NTOK, WD), jnp.float32),
      scratch_types=[
          pltpu.VMEM((W_NCH, W_CHUNK), jnp.int32),
          pltpu.VMEM((W_CHUNK, WD), jnp.float32),
          pltpu.SemaphoreType.DMA,
      ],
  )(_sc_word_body)
  return fn(w_idx3d, word_emb)


def _sc_vu_body(v_idx, v_tab, u_idx, u_tab, v_out, u_out, idx_v, buf, sem):
  c = lax.axis_index("c")
  s = lax.axis_index("s")
  wid = s * _NC + c

  def one_chunk(idx_hbm, tab, out, chunk):
    pltpu.sync_copy(idx_hbm.at[chunk], idx_v)
    pltpu.async_copy(tab.at[idx_v.at[0]], buf, sem).wait()
    pltpu.sync_copy(buf, out.at[pl.ds(chunk * V_CHUNK, V_CHUNK)])

  one_chunk(v_idx, v_tab, v_out, wid)

  @pl.when(wid < V_NCH - NW)
  def _():
    one_chunk(v_idx, v_tab, v_out, wid + NW)

  @pl.when(jnp.logical_and(wid >= NW - U_NCH, wid < NW))
  def _():
    one_chunk(u_idx, u_tab, u_out, wid - (NW - U_NCH))


def _sc_vu(v_idx3d, v_tab, u_idx3d, u_tab):
  mesh = plsc.VectorSubcoreMesh(core_axis_name="c", subcore_axis_name="s")
  fn = functools.partial(
      pl.kernel,
      mesh=mesh,
      out_type=[
          jax.ShapeDtypeStruct((NSEQ, EP), jnp.float32),
          jax.ShapeDtypeStruct((B, EP), jnp.float32),
      ],
      scratch_types=[
          pltpu.VMEM((1, V_CHUNK), jnp.int32),
          pltpu.VMEM((V_CHUNK, EP), jnp.float32),
          pltpu.SemaphoreType.DMA,
      ],
  )(_sc_vu_body)
  return fn(v_idx3d, v_tab, u_idx3d, u_tab)


def _pad_body(v_ref, u_ref, vo_ref, uo_ref):
  z = jnp.zeros((PAD_ROWS, EP - E), jnp.float32)
  vo_ref[...] = jnp.concatenate([v_ref[...], z], axis=1)
  uo_ref[...] = jnp.concatenate([u_ref[...], z], axis=1)


def _pad_tables(v2e_w, u2e_w):
  return pl.pallas_call(
      _pad_body,
      grid=(PAD_GRID,),
      in_specs=[
          pl.BlockSpec((PAD_ROWS, E), lambda i: (i, 0)),
          pl.BlockSpec((PAD_ROWS, E), lambda i: (i, 0)),
      ],
      out_specs=[
          pl.BlockSpec((PAD_ROWS, EP), lambda i: (i, 0)),
          pl.BlockSpec((PAD_ROWS, EP), lambda i: (i, 0)),
      ],
      out_shape=[
          jax.ShapeDtypeStruct((NV, EP), jnp.float32),
          jax.ShapeDtypeStruct((NV, EP), jnp.float32),
      ],
  )(v2e_w, u2e_w)


def _tc_body(emb_ref, euv_ref, u_ref, r_ref, r2e_ref,
             w3_ref, bc3_ref, w5_ref, bc5_ref,
             w1a_ref, w1b_ref, w1c3_ref, w1c5_ref, b1_ref,
             w2_ref, b2_ref,
             a1o_ref, a1u_ref, a1b_ref, a2_ref, a2b_ref, a3_ref, a3b_ref,
             out_ref):
  f32 = jnp.float32
  dot = lambda a, b: lax.dot(a, b, preferred_element_type=f32)
  relu = lambda x: jnp.maximum(x, 0.0)

  emb = emb_ref[...]

  w3 = w3_ref[...]
  acc3 = dot(emb[:, 0:3 * WD], w3)
  for l in range(1, SEQ - 3 + 1):
    acc3 = jnp.maximum(acc3, dot(emb[:, l * WD:(l + 3) * WD], w3))
  e3 = relu(acc3 + bc3_ref[...])

  w5 = w5_ref[...]
  acc5 = dot(emb[:, 0:5 * WD], w5)
  for l in range(1, SEQ - 5 + 1):
    acc5 = jnp.maximum(acc5, dot(emb[:, l * WD:(l + 5) * WD], w5))
  e5 = relu(acc5 + bc5_ref[...])

  euv = euv_ref[...]
  r = r_ref[...]
  onehot = (lax.broadcasted_iota(jnp.int32, (NSB, 8), 1) == r).astype(f32)
  rmat = dot(r2e_ref[...], w1b_ref[...])
  h = (dot(euv, w1a_ref[...]) + dot(onehot, rmat)
       + dot(e3, w1c3_ref[...]) + dot(e5, w1c5_ref[...]) + b1_ref[...])
  h = relu(h)
  o = relu(dot(h, w2_ref[...]) + b2_ref[...])

  u = u_ref[...]
  ua = dot(u, a1u_ref[...])
  ua_b = jnp.broadcast_to(ua[:, None, :], (NB, HIST, E)).reshape(NSB, E)
  a = relu(dot(o, a1o_ref[...]) + ua_b + a1b_ref[...])
  a = relu(dot(a, a2_ref[...]) + a2b_ref[...])
  s = dot(a, a3_ref[...]) + a3b_ref[...]

  srs = s.reshape(NB, HIST)
  m = jnp.max(srs, axis=1, keepdims=True)
  ex = jnp.exp(srs - m)
  wgt = ex / jnp.sum(ex, axis=1, keepdims=True)

  o3 = o.reshape(NB, HIST, E)
  out_ref[...] = jnp.sum(o3 * wgt[:, :, None], axis=1)


def _full_spec(shape):
  nd = len(shape)
  return pl.BlockSpec(shape, lambda i, _n=nd: (0,) * _n)


def kernel(nodes, history_uv, history_r, history_w,
           v2e_w, u2e_w, r2e_w, word_emb,
           Wc3, bc3, Wc5, bc5,
           W1, b1, W2, b2,
           A1, a1, A2, a2, A3, a3):
  f32 = jnp.float32
  w_idx3d = history_w.astype(jnp.int32).reshape(NW, W_NCH, W_CHUNK)
  v_idx3d = history_uv.astype(jnp.int32).reshape(V_NCH, 1, V_CHUNK)
  u_idx3d = nodes.astype(jnp.int32).reshape(U_NCH, 1, V_CHUNK)

  w_rows = _sc_word(w_idx3d, word_emb)
  v_pad, u_pad = _pad_tables(v2e_w, u2e_w)
  v_rows, u_rows = _sc_vu(v_idx3d, v_pad, u_idx3d, u_pad)
  emb = w_rows.reshape(NSEQ, SEQ * WD)

  w3 = Wc3.reshape(NF, 3 * WD).T
  w5 = Wc5.reshape(NF, 5 * WD).T
  w1t = W1.T
  w1a = jnp.pad(w1t[:E], ((0, EP - E), (0, 0)))
  w1b = w1t[E:2 * E]
  w1c3 = w1t[2 * E:2 * E + NF]
  w1c5 = w1t[2 * E + NF:]
  r2e8 = jnp.zeros((8, E), f32).at[:6].set(r2e_w)
  a1t = A1.T
  a1o = a1t[:E]
  a1u = jnp.pad(a1t[E:], ((0, EP - E), (0, 0)))

  grid_spec = pl.GridSpec(
      grid=(GRID,),
      in_specs=[
          pl.BlockSpec((NSB, SEQ * WD), lambda i: (i, 0)),
          pl.BlockSpec((NSB, EP), lambda i: (i, 0)),
          pl.BlockSpec((NB, EP), lambda i: (i, 0)),
          pl.BlockSpec((NSB, 1), lambda i: (i, 0)),
          _full_spec((8, E)),
          _full_spec((3 * WD, NF)),
          _full_spec((1, NF)),
          _full_spec((5 * WD, NF)),
          _full_spec((1, NF)),
          _full_spec((EP, E)),
          _full_spec((E, E)),
          _full_spec((NF, E)),
          _full_spec((NF, E)),
          _full_spec((1, E)),
          _full_spec((E, E)),
          _full_spec((1, E)),
          _full_spec((E, E)),
          _full_spec((EP, E)),
          _full_spec((1, E)),
          _full_spec((E, E)),
          _full_spec((1, E)),
          _full_spec((E, 1)),
          _full_spec((1, 1)),
      ],
      out_specs=pl.BlockSpec((NB, E), lambda i: (i, 0)),
  )

  out = pl.pallas_call(
      _tc_body,
      grid_spec=grid_spec,
      out_shape=jax.ShapeDtypeStruct((B, E), f32),
  )(
      emb, v_rows, u_rows, history_r.astype(jnp.int32).reshape(NSEQ, 1), r2e8,
      w3, bc3.reshape(1, NF), w5, bc5.reshape(1, NF),
      w1a, w1b, w1c3, w1c5, b1.reshape(1, E),
      W2.T, b2.reshape(1, E),
      a1o, a1u, a1.reshape(1, E), A2.T, a2.reshape(1, E),
      A3.T, a3.reshape(1, 1),
  )
  return out


# speedup vs baseline: 5.7623x; 1.3907x over previous
import functools

import jax
import jax.numpy as jnp
from jax import lax
from jax.experimental import pallas as pl
from jax.experimental.pallas import tpu as pltpu
from jax.experimental.pallas import tpu_sc as plsc

B = 256
HIST = 20
SEQ = 20
E = 100
WD = 64
NF = 100
NV = 100000
NSEQ = B * HIST
NTOK = NSEQ * SEQ
EP = 128

_NC = 2
_NS = 16
NW = _NC * _NS

W_CHUNK = 128
W_PER = NTOK // NW
W_NCH = W_PER // W_CHUNK
V_CHUNK = 128
V_NCH = NSEQ // V_CHUNK
U_NCH = B // V_CHUNK

NB = 32
GRID = B // NB
NSB = NB * HIST

PAD_ROWS = 12800
PAD_GRID = 8
NVP = PAD_ROWS * PAD_GRID


def _sc_word_body(w_idx, w_tab, w_out, widx_v, wbufa, wbufb, sema, semb):
  c = lax.axis_index("c")
  s = lax.axis_index("s")
  wid = s * _NC + c
  pltpu.sync_copy(w_idx.at[wid], widx_v)
  base = wid * W_PER

  def gather(j, buf, sem):
    return pltpu.async_copy(w_tab.at[widx_v.at[j]], buf, sem)

  gather(0, wbufa, sema)

  def w_body(i, carry):
    ja = 2 * i
    gather(ja + 1, wbufb, semb)
    pltpu.make_async_copy(w_tab.at[widx_v.at[ja]], wbufa, sema).wait()
    pltpu.sync_copy(wbufa, w_out.at[pl.ds(base + ja * W_CHUNK, W_CHUNK)])
    gather(ja + 2, wbufa, sema)
    pltpu.make_async_copy(w_tab.at[widx_v.at[ja]], wbufb, semb).wait()
    pltpu.sync_copy(wbufb, w_out.at[pl.ds(base + (ja + 1) * W_CHUNK, W_CHUNK)])
    return carry

  lax.fori_loop(0, (W_NCH - 1) // 2, w_body, 0)
  pltpu.make_async_copy(w_tab.at[widx_v.at[0]], wbufa, sema).wait()
  pltpu.sync_copy(wbufa, w_out.at[pl.ds(base + (W_NCH - 1) * W_CHUNK, W_CHUNK)])


def _sc_word(w_idx3d, word_emb):
  mesh = plsc.VectorSubcoreMesh(core_axis_name="c", subcore_axis_name="s")
  fn = functools.partial(
      pl.kernel,
      mesh=mesh,
      compiler_params=pltpu.CompilerParams(use_tc_tiling_on_sc=False),
      out_type=jax.ShapeDtypeStruct((NTOK, WD), jnp.float32),
      scratch_types=[
          pltpu.VMEM((W_NCH, W_CHUNK), jnp.int32),
          pltpu.VMEM((W_CHUNK, WD), jnp.float32),
          pltpu.VMEM((W_CHUNK, WD), jnp.float32),
          pltpu.SemaphoreType.DMA,
          pltpu.SemaphoreType.DMA,
      ],
  )(_sc_word_body)
  return fn(w_idx3d, word_emb)


def _sc_vu_body(v_idx, v_tab, u_idx, u_tab, v_out, u_out, idx_v, buf, sem):
  c = lax.axis_index("c")
  s = lax.axis_index("s")
  wid = s * _NC + c

  def one_chunk(idx_hbm, tab, out, chunk):
    pltpu.sync_copy(idx_hbm.at[chunk], idx_v)
    pltpu.async_copy(tab.at[idx_v.at[0]], buf, sem).wait()
    pltpu.sync_copy(buf, out.at[pl.ds(chunk * V_CHUNK, V_CHUNK)])

  one_chunk(v_idx, v_tab, v_out, wid)

  @pl.when(wid < V_NCH - NW)
  def _():
    one_chunk(v_idx, v_tab, v_out, wid + NW)

  @pl.when(jnp.logical_and(wid >= NW - U_NCH, wid < NW))
  def _():
    one_chunk(u_idx, u_tab, u_out, wid - (NW - U_NCH))


def _sc_vu(v_idx3d, v_tab, u_idx3d, u_tab):
  mesh = plsc.VectorSubcoreMesh(core_axis_name="c", subcore_axis_name="s")
  fn = functools.partial(
      pl.kernel,
      mesh=mesh,
      out_type=[
          jax.ShapeDtypeStruct((NSEQ, EP), jnp.float32),
          jax.ShapeDtypeStruct((B, EP), jnp.float32),
      ],
      name="sc_vu_gather",
      scratch_types=[
          pltpu.VMEM((1, V_CHUNK), jnp.int32),
          pltpu.VMEM((V_CHUNK, EP), jnp.float32),
          pltpu.SemaphoreType.DMA,
      ],
  )(_sc_vu_body)
  return fn(v_idx3d, v_tab, u_idx3d, u_tab)


def _proj_body(vt_ref, ut_ref, wv_ref, wu_ref, vo_ref, uo_ref):
  dn = (((0,), (0,)), ((), ()))
  vo_ref[...] = lax.dot_general(vt_ref[...], wv_ref[...], dn,
                                preferred_element_type=jnp.float32)
  uo_ref[...] = lax.dot_general(ut_ref[...], wu_ref[...], dn,
                                preferred_element_type=jnp.float32)


def _proj_tables(v2e_t, u2e_t, wv, wu):
  return pl.pallas_call(
      _proj_body,
      grid=(PAD_GRID,),
      in_specs=[
          pl.BlockSpec((E, PAD_ROWS), lambda i: (0, i)),
          pl.BlockSpec((E, PAD_ROWS), lambda i: (0, i)),
          _full_spec((E, EP)),
          _full_spec((E, EP)),
      ],
      out_specs=[
          pl.BlockSpec((PAD_ROWS, EP), lambda i: (i, 0)),
          pl.BlockSpec((PAD_ROWS, EP), lambda i: (i, 0)),
      ],
      out_shape=[
          jax.ShapeDtypeStruct((NVP, EP), jnp.float32),
          jax.ShapeDtypeStruct((NVP, EP), jnp.float32),
      ],
  )(v2e_t, u2e_t, wv, wu)


def _tc_body(emb_ref, euv_ref, u_ref, r_ref, r2e_ref,
             w3_ref, bc3_ref, w5_ref, bc5_ref,
             w1b_ref, w1c3_ref, w1c5_ref, b1_ref,
             w2_ref, b2_ref,
             a1o_ref, a1b_ref, a2_ref, a2b_ref, a3_ref, a3b_ref,
             out_ref):
  f32 = jnp.float32
  dot = lambda a, b: lax.dot(a, b, preferred_element_type=f32)
  relu = lambda x: jnp.maximum(x, 0.0)

  emb = emb_ref[...]

  w3 = w3_ref[...]
  acc3 = dot(emb[:, 0:3 * WD], w3)
  for l in range(1, SEQ - 3 + 1):
    acc3 = jnp.maximum(acc3, dot(emb[:, l * WD:(l + 3) * WD], w3))
  e3 = relu(acc3 + bc3_ref[...])

  w5 = w5_ref[...]
  acc5 = dot(emb[:, 0:5 * WD], w5)
  for l in range(1, SEQ - 5 + 1):
    acc5 = jnp.maximum(acc5, dot(emb[:, l * WD:(l + 5) * WD], w5))
  e5 = relu(acc5 + bc5_ref[...])

  euv_p = euv_ref[...][:, :E]
  r = r_ref[...]
  onehot = (lax.broadcasted_iota(jnp.int32, (NSB, 8), 1) == r).astype(f32)
  rmat = dot(r2e_ref[...], w1b_ref[...])
  h = (euv_p + dot(onehot, rmat)
       + dot(e3, w1c3_ref[...]) + dot(e5, w1c5_ref[...]) + b1_ref[...])
  h = relu(h)
  o = relu(dot(h, w2_ref[...]) + b2_ref[...])

  ua = u_ref[...][:, :E]
  ua_b = jnp.broadcast_to(ua[:, None, :], (NB, HIST, E)).reshape(NSB, E)
  a = relu(dot(o, a1o_ref[...]) + ua_b + a1b_ref[...])
  a = relu(dot(a, a2_ref[...]) + a2b_ref[...])
  s = dot(a, a3_ref[...]) + a3b_ref[...]

  srs = s.reshape(NB, HIST)
  m = jnp.max(srs, axis=1, keepdims=True)
  ex = jnp.exp(srs - m)
  wgt = ex / jnp.sum(ex, axis=1, keepdims=True)

  o3 = o.reshape(NB, HIST, E)
  out_ref[...] = jnp.sum(o3 * wgt[:, :, None], axis=1)


def _full_spec(shape):
  nd = len(shape)
  return pl.BlockSpec(shape, lambda i, _n=nd: (0,) * _n)


def kernel(nodes, history_uv, history_r, history_w,
           v2e_w, u2e_w, r2e_w, word_emb,
           Wc3, bc3, Wc5, bc5,
           W1, b1, W2, b2,
           A1, a1, A2, a2, A3, a3):
  f32 = jnp.float32
  w_idx3d = history_w.astype(jnp.int32).reshape(NW, W_NCH, W_CHUNK)
  v_idx3d = history_uv.astype(jnp.int32).reshape(V_NCH, 1, V_CHUNK)
  u_idx3d = nodes.astype(jnp.int32).reshape(U_NCH, 1, V_CHUNK)

  w3 = Wc3.reshape(NF, 3 * WD).T
  w5 = Wc5.reshape(NF, 5 * WD).T
  w1t = W1.T
  w1a = jnp.pad(w1t[:E], ((0, 0), (0, EP - E)))
  w1b = w1t[E:2 * E]
  w1c3 = w1t[2 * E:2 * E + NF]
  w1c5 = w1t[2 * E + NF:]
  r2e8 = jnp.zeros((8, E), f32).at[:6].set(r2e_w)
  a1t = A1.T
  a1o = a1t[:E]
  a1u = jnp.pad(a1t[E:], ((0, 0), (0, EP - E)))

  w_rows = _sc_word(w_idx3d, word_emb)
  v_proj, u_proj = _proj_tables(v2e_w.T, u2e_w.T, w1a, a1u)
  v_rows, u_rows = _sc_vu(v_idx3d, v_proj, u_idx3d, u_proj)
  emb = w_rows.reshape(NSEQ, SEQ * WD)

  grid_spec = pl.GridSpec(
      grid=(GRID,),
      in_specs=[
          pl.BlockSpec((NSB, SEQ * WD), lambda i: (i, 0)),
          pl.BlockSpec((NSB, EP), lambda i: (i, 0)),
          pl.BlockSpec((NB, EP), lambda i: (i, 0)),
          pl.BlockSpec((NSB, 1), lambda i: (i, 0)),
          _full_spec((8, E)),
          _full_spec((3 * WD, NF)),
          _full_spec((1, NF)),
          _full_spec((5 * WD, NF)),
          _full_spec((1, NF)),
          _full_spec((E, E)),
          _full_spec((NF, E)),
          _full_spec((NF, E)),
          _full_spec((1, E)),
          _full_spec((E, E)),
          _full_spec((1, E)),
          _full_spec((E, E)),
          _full_spec((1, E)),
          _full_spec((E, E)),
          _full_spec((1, E)),
          _full_spec((E, 1)),
          _full_spec((1, 1)),
      ],
      out_specs=pl.BlockSpec((NB, E), lambda i: (i, 0)),
  )

  out = pl.pallas_call(
      _tc_body,
      grid_spec=grid_spec,
      out_shape=jax.ShapeDtypeStruct((B, E), f32),
  )(
      emb, v_rows, u_rows, history_r.astype(jnp.int32).reshape(NSEQ, 1), r2e8,
      w3, bc3.reshape(1, NF), w5, bc5.reshape(1, NF),
      w1b, w1c3, w1c5, b1.reshape(1, E),
      W2.T, b2.reshape(1, E),
      a1o, a1.reshape(1, E), A2.T, a2.reshape(1, E),
      A3.T, a3.reshape(1, 1),
  )
  return out


# speedup vs baseline: 5.9777x; 1.0374x over previous
import functools

import jax
import jax.numpy as jnp
from jax import lax
from jax.experimental import pallas as pl
from jax.experimental.pallas import tpu as pltpu
from jax.experimental.pallas import tpu_sc as plsc

B = 256
HIST = 20
SEQ = 20
E = 100
WD = 64
NF = 100
NV = 100000
NSEQ = B * HIST
NTOK = NSEQ * SEQ
EP = 128

_NC = 2
_NS = 16
NW = _NC * _NS

W_CHUNK = 128
W_PER = NTOK // NW
W_NCH = W_PER // W_CHUNK
V_CHUNK = 128
V_NCH = NSEQ // V_CHUNK
U_NCH = B // V_CHUNK

NB = 32
GRID = B // NB
NSB = NB * HIST

PAD_ROWS = 12800
PAD_GRID = 8
NVP = PAD_ROWS * PAD_GRID


def _sc_word_body(w_idx, w_tab, w_out, widx_v, wbufa, wbufb, sema, semb):
  c = lax.axis_index("c")
  s = lax.axis_index("s")
  wid = s * _NC + c
  pltpu.sync_copy(w_idx.at[wid], widx_v)
  base = wid * W_PER

  def gather(j, buf, sem):
    return pltpu.async_copy(w_tab.at[widx_v.at[j]], buf, sem)

  gather(0, wbufa, sema)

  def w_body(i, carry):
    ja = 2 * i
    gather(ja + 1, wbufb, semb)
    pltpu.make_async_copy(w_tab.at[widx_v.at[ja]], wbufa, sema).wait()
    pltpu.sync_copy(wbufa, w_out.at[pl.ds(base + ja * W_CHUNK, W_CHUNK)])
    gather(ja + 2, wbufa, sema)
    pltpu.make_async_copy(w_tab.at[widx_v.at[ja]], wbufb, semb).wait()
    pltpu.sync_copy(wbufb, w_out.at[pl.ds(base + (ja + 1) * W_CHUNK, W_CHUNK)])
    return carry

  lax.fori_loop(0, (W_NCH - 1) // 2, w_body, 0)
  pltpu.make_async_copy(w_tab.at[widx_v.at[0]], wbufa, sema).wait()
  pltpu.sync_copy(wbufa, w_out.at[pl.ds(base + (W_NCH - 1) * W_CHUNK, W_CHUNK)])


def _sc_word(w_idx3d, word_emb):
  mesh = plsc.VectorSubcoreMesh(core_axis_name="c", subcore_axis_name="s")
  fn = functools.partial(
      pl.kernel,
      mesh=mesh,
      compiler_params=pltpu.CompilerParams(use_tc_tiling_on_sc=False),
      out_type=jax.ShapeDtypeStruct((NTOK, WD), jnp.float32),
      scratch_types=[
          pltpu.VMEM((W_NCH, W_CHUNK), jnp.int32),
          pltpu.VMEM((W_CHUNK, WD), jnp.float32),
          pltpu.VMEM((W_CHUNK, WD), jnp.float32),
          pltpu.SemaphoreType.DMA,
          pltpu.SemaphoreType.DMA,
      ],
  )(_sc_word_body)
  return fn(w_idx3d, word_emb)


def _sc_v_body(v_idx, v_tab, v_out, idx_v, buf, sem):
  c = lax.axis_index("c")
  s = lax.axis_index("s")
  wid = s * _NC + c

  def one_chunk(chunk):
    pltpu.sync_copy(v_idx.at[chunk], idx_v)
    pltpu.async_copy(v_tab.at[idx_v.at[0]], buf, sem).wait()
    pltpu.sync_copy(buf, v_out.at[pl.ds(chunk * V_CHUNK, V_CHUNK)])

  one_chunk(wid)

  @pl.when(wid < V_NCH - NW)
  def _():
    one_chunk(wid + NW)


def _sc_v(v_idx3d, v_tab):
  mesh = plsc.VectorSubcoreMesh(core_axis_name="c", subcore_axis_name="s")
  fn = functools.partial(
      pl.kernel,
      mesh=mesh,
      out_type=jax.ShapeDtypeStruct((NSEQ, EP), jnp.float32),
      name="sc_v_gather",
      scratch_types=[
          pltpu.VMEM((1, V_CHUNK), jnp.int32),
          pltpu.VMEM((V_CHUNK, EP), jnp.float32),
          pltpu.SemaphoreType.DMA,
      ],
  )(_sc_v_body)
  return fn(v_idx3d, v_tab)


def _proj_body(vt_ref, wv_ref, vo_ref):
  dn = (((0,), (0,)), ((), ()))
  vo_ref[...] = lax.dot_general(vt_ref[...], wv_ref[...], dn,
                                preferred_element_type=jnp.float32)


def _proj_table(v2e_t, wv):
  return pl.pallas_call(
      _proj_body,
      grid=(PAD_GRID,),
      in_specs=[
          pl.BlockSpec((E, PAD_ROWS), lambda i: (0, i)),
          _full_spec((E, EP)),
      ],
      out_specs=pl.BlockSpec((PAD_ROWS, EP), lambda i: (i, 0)),
      out_shape=jax.ShapeDtypeStruct((NVP, EP), jnp.float32),
  )(v2e_t, wv)


def _ugather_body(ut_ref, nodes_ref, out_ref):
  p = pl.program_id(0)
  base = p * PAD_ROWS
  cols = lax.broadcasted_iota(jnp.int32, (1, PAD_ROWS), 1) + base
  ut = jnp.where(cols < NV, ut_ref[...], 0.0)
  onehot = (jnp.broadcast_to(cols, (B, PAD_ROWS))
            == nodes_ref[...]).astype(jnp.float32)
  part = lax.dot_general(onehot, ut, (((1,), (1,)), ((), ())),
                         preferred_element_type=jnp.float32)

  @pl.when(p == 0)
  def _():
    out_ref[...] = jnp.zeros_like(out_ref)

  out_ref[...] += part


def _ugather(u2e_t, nodes):
  return pl.pallas_call(
      _ugather_body,
      grid=(PAD_GRID,),
      in_specs=[
          pl.BlockSpec((E, PAD_ROWS), lambda i: (0, i)),
          _full_spec((B, 1)),
      ],
      out_specs=pl.BlockSpec((B, E), lambda i: (0, 0)),
      out_shape=jax.ShapeDtypeStruct((B, E), jnp.float32),
  )(u2e_t, nodes)


def _tc_body(emb_ref, euv_ref, u_ref, r_ref, r2e_ref,
             w3_ref, bc3_ref, w5_ref, bc5_ref,
             w1b_ref, w1c3_ref, w1c5_ref, b1_ref,
             w2_ref, b2_ref,
             a1o_ref, a1u_ref, a1b_ref, a2_ref, a2b_ref, a3_ref, a3b_ref,
             out_ref):
  f32 = jnp.float32
  dot = lambda a, b: lax.dot(a, b, preferred_element_type=f32)
  relu = lambda x: jnp.maximum(x, 0.0)

  emb = emb_ref[...]

  w3 = w3_ref[...]
  acc3 = dot(emb[:, 0:3 * WD], w3)
  for l in range(1, SEQ - 3 + 1):
    acc3 = jnp.maximum(acc3, dot(emb[:, l * WD:(l + 3) * WD], w3))
  e3 = relu(acc3 + bc3_ref[...])

  w5 = w5_ref[...]
  acc5 = dot(emb[:, 0:5 * WD], w5)
  for l in range(1, SEQ - 5 + 1):
    acc5 = jnp.maximum(acc5, dot(emb[:, l * WD:(l + 5) * WD], w5))
  e5 = relu(acc5 + bc5_ref[...])

  euv_p = euv_ref[...][:, :E]
  r = r_ref[...]
  onehot = (lax.broadcasted_iota(jnp.int32, (NSB, 8), 1) == r).astype(f32)
  rmat = dot(r2e_ref[...], w1b_ref[...])
  h = (euv_p + dot(onehot, rmat)
       + dot(e3, w1c3_ref[...]) + dot(e5, w1c5_ref[...]) + b1_ref[...])
  h = relu(h)
  o = relu(dot(h, w2_ref[...]) + b2_ref[...])

  ua = dot(u_ref[...], a1u_ref[...])
  ua_b = jnp.broadcast_to(ua[:, None, :], (NB, HIST, E)).reshape(NSB, E)
  a = relu(dot(o, a1o_ref[...]) + ua_b + a1b_ref[...])
  a = relu(dot(a, a2_ref[...]) + a2b_ref[...])
  s = dot(a, a3_ref[...]) + a3b_ref[...]

  srs = s.reshape(NB, HIST)
  m = jnp.max(srs, axis=1, keepdims=True)
  ex = jnp.exp(srs - m)
  wgt = ex / jnp.sum(ex, axis=1, keepdims=True)

  o3 = o.reshape(NB, HIST, E)
  out_ref[...] = jnp.sum(o3 * wgt[:, :, None], axis=1)


def _full_spec(shape):
  nd = len(shape)
  return pl.BlockSpec(shape, lambda i, _n=nd: (0,) * _n)


def kernel(nodes, history_uv, history_r, history_w,
           v2e_w, u2e_w, r2e_w, word_emb,
           Wc3, bc3, Wc5, bc5,
           W1, b1, W2, b2,
           A1, a1, A2, a2, A3, a3):
  f32 = jnp.float32
  w_idx3d = history_w.astype(jnp.int32).reshape(NW, W_NCH, W_CHUNK)
  v_idx3d = history_uv.astype(jnp.int32).reshape(V_NCH, 1, V_CHUNK)
  u_idx3d = nodes.astype(jnp.int32).reshape(U_NCH, 1, V_CHUNK)

  w3 = Wc3.reshape(NF, 3 * WD).T
  w5 = Wc5.reshape(NF, 5 * WD).T
  w1t = W1.T
  w1a = jnp.pad(w1t[:E], ((0, 0), (0, EP - E)))
  w1b = w1t[E:2 * E]
  w1c3 = w1t[2 * E:2 * E + NF]
  w1c5 = w1t[2 * E + NF:]
  r2e8 = jnp.zeros((8, E), f32).at[:6].set(r2e_w)
  a1t = A1.T
  a1o = a1t[:E]
  a1u = a1t[E:]

  w_rows = _sc_word(w_idx3d, word_emb)
  v_proj = _proj_table(v2e_w.T, w1a)
  v_rows = _sc_v(v_idx3d, v_proj)
  u_rows = _ugather(u2e_w.T, nodes.astype(jnp.int32).reshape(B, 1))
  emb = w_rows.reshape(NSEQ, SEQ * WD)

  grid_spec = pl.GridSpec(
      grid=(GRID,),
      in_specs=[
          pl.BlockSpec((NSB, SEQ * WD), lambda i: (i, 0)),
          pl.BlockSpec((NSB, EP), lambda i: (i, 0)),
          pl.BlockSpec((NB, E), lambda i: (i, 0)),
          pl.BlockSpec((NSB, 1), lambda i: (i, 0)),
          _full_spec((8, E)),
          _full_spec((3 * WD, NF)),
          _full_spec((1, NF)),
          _full_spec((5 * WD, NF)),
          _full_spec((1, NF)),
          _full_spec((E, E)),
          _full_spec((NF, E)),
          _full_spec((NF, E)),
          _full_spec((1, E)),
          _full_spec((E, E)),
          _full_spec((1, E)),
          _full_spec((E, E)),
          _full_spec((E, E)),
          _full_spec((1, E)),
          _full_spec((E, E)),
          _full_spec((1, E)),
          _full_spec((E, 1)),
          _full_spec((1, 1)),
      ],
      out_specs=pl.BlockSpec((NB, E), lambda i: (i, 0)),
  )

  out = pl.pallas_call(
      _tc_body,
      grid_spec=grid_spec,
      out_shape=jax.ShapeDtypeStruct((B, E), f32),
  )(
      emb, v_rows, u_rows, history_r.astype(jnp.int32).reshape(NSEQ, 1), r2e8,
      w3, bc3.reshape(1, NF), w5, bc5.reshape(1, NF),
      w1b, w1c3, w1c5, b1.reshape(1, E),
      W2.T, b2.reshape(1, E),
      a1o, a1u, a1.reshape(1, E), A2.T, a2.reshape(1, E),
      A3.T, a3.reshape(1, 1),
  )
  return out


# speedup vs baseline: 6.1726x; 1.0326x over previous
import functools

import jax
import jax.numpy as jnp
from jax import lax
from jax.experimental import pallas as pl
from jax.experimental.pallas import tpu as pltpu
from jax.experimental.pallas import tpu_sc as plsc

B = 256
HIST = 20
SEQ = 20
E = 100
WD = 64
NF = 100
NV = 100000
NSEQ = B * HIST
NTOK = NSEQ * SEQ
EP = 128

_NC = 2
_NS = 16
NW = _NC * _NS

W_CHUNK = 128
W_PER = NTOK // NW
W_NCH = W_PER // W_CHUNK
V_CHUNK = 128
V_NCH = NSEQ // V_CHUNK
U_NCH = B // V_CHUNK

NB = 32
GRID = B // NB
NSB = NB * HIST

PAD_ROWS = 12800
PAD_GRID = 8
NVP = PAD_ROWS * PAD_GRID


def _sc_word_body(w_idx, w_tab, w_out, widx_v, wbufa, wbufb,
                  sema, semb, semsa, semsb):
  c = lax.axis_index("c")
  s = lax.axis_index("s")
  wid = s * _NC + c
  pltpu.sync_copy(w_idx.at[wid], widx_v)
  base = wid * W_PER

  def gather(j, buf, sem):
    pltpu.async_copy(w_tab.at[widx_v.at[j]], buf, sem)

  def gwait(buf, sem):
    pltpu.make_async_copy(w_tab.at[widx_v.at[0]], buf, sem).wait()

  def store(j, buf, sem):
    pltpu.async_copy(buf, w_out.at[pl.ds(base + j * W_CHUNK, W_CHUNK)], sem)

  def swait(j, buf, sem):
    pltpu.make_async_copy(
        buf, w_out.at[pl.ds(base + j * W_CHUNK, W_CHUNK)], sem).wait()

  gather(0, wbufa, sema)
  gather(1, wbufb, semb)

  def w_body(i, carry):
    ja = 2 * i
    gwait(wbufa, sema)
    store(ja, wbufa, semsa)
    gwait(wbufb, semb)
    store(ja + 1, wbufb, semsb)
    swait(ja, wbufa, semsa)

    @pl.when(ja + 2 < W_NCH)
    def _():
      gather(ja + 2, wbufa, sema)

    swait(ja + 1, wbufb, semsb)

    @pl.when(ja + 3 < W_NCH)
    def _():
      gather(ja + 3, wbufb, semb)
    return carry

  lax.fori_loop(0, W_NCH // 2, w_body, 0)
  gwait(wbufa, sema)
  pltpu.sync_copy(wbufa, w_out.at[pl.ds(base + (W_NCH - 1) * W_CHUNK, W_CHUNK)])


def _sc_word(w_idx3d, word_emb):
  mesh = plsc.VectorSubcoreMesh(core_axis_name="c", subcore_axis_name="s")
  fn = functools.partial(
      pl.kernel,
      mesh=mesh,
      compiler_params=pltpu.CompilerParams(use_tc_tiling_on_sc=False),
      out_type=jax.ShapeDtypeStruct((NTOK, WD), jnp.float32),
      scratch_types=[
          pltpu.VMEM((W_NCH, W_CHUNK), jnp.int32),
          pltpu.VMEM((W_CHUNK, WD), jnp.float32),
          pltpu.VMEM((W_CHUNK, WD), jnp.float32),
          pltpu.SemaphoreType.DMA,
          pltpu.SemaphoreType.DMA,
          pltpu.SemaphoreType.DMA,
          pltpu.SemaphoreType.DMA,
      ],
  )(_sc_word_body)
  return fn(w_idx3d, word_emb)


def _sc_v_body(v_idx, v_tab, v_out, idx_v, buf, sem):
  c = lax.axis_index("c")
  s = lax.axis_index("s")
  wid = s * _NC + c

  def one_chunk(chunk):
    pltpu.sync_copy(v_idx.at[chunk], idx_v)
    pltpu.async_copy(v_tab.at[idx_v.at[0]], buf, sem).wait()
    pltpu.sync_copy(buf, v_out.at[pl.ds(chunk * V_CHUNK, V_CHUNK)])

  one_chunk(wid)

  @pl.when(wid < V_NCH - NW)
  def _():
    one_chunk(wid + NW)


def _sc_v(v_idx3d, v_tab):
  mesh = plsc.VectorSubcoreMesh(core_axis_name="c", subcore_axis_name="s")
  fn = functools.partial(
      pl.kernel,
      mesh=mesh,
      out_type=jax.ShapeDtypeStruct((NSEQ, EP), jnp.float32),
      name="sc_v_gather",
      scratch_types=[
          pltpu.VMEM((1, V_CHUNK), jnp.int32),
          pltpu.VMEM((V_CHUNK, EP), jnp.float32),
          pltpu.SemaphoreType.DMA,
      ],
  )(_sc_v_body)
  return fn(v_idx3d, v_tab)


def _proj_body(vt_ref, ut_ref, wv_ref, nodes_ref, vo_ref, uo_ref):
  p = pl.program_id(0)
  dn0 = (((0,), (0,)), ((), ()))
  vo_ref[...] = lax.dot_general(vt_ref[...], wv_ref[...], dn0,
                                preferred_element_type=jnp.float32)

  base = p * PAD_ROWS
  cols = lax.broadcasted_iota(jnp.int32, (1, PAD_ROWS), 1) + base
  ut = jnp.where(cols < NV, ut_ref[...], 0.0)
  onehot = (jnp.broadcast_to(cols, (B, PAD_ROWS))
            == nodes_ref[...]).astype(jnp.float32)
  part = lax.dot_general(onehot, ut, (((1,), (1,)), ((), ())),
                         preferred_element_type=jnp.float32)

  @pl.when(p == 0)
  def _():
    uo_ref[...] = jnp.zeros_like(uo_ref)

  uo_ref[...] += part


def _proj_tables(v2e_t, u2e_t, wv, nodes):
  return pl.pallas_call(
      _proj_body,
      grid=(PAD_GRID,),
      in_specs=[
          pl.BlockSpec((E, PAD_ROWS), lambda i: (0, i)),
          pl.BlockSpec((E, PAD_ROWS), lambda i: (0, i)),
          _full_spec((E, EP)),
          _full_spec((B, 1)),
      ],
      out_specs=[
          pl.BlockSpec((PAD_ROWS, EP), lambda i: (i, 0)),
          pl.BlockSpec((B, E), lambda i: (0, 0)),
      ],
      out_shape=[
          jax.ShapeDtypeStruct((NVP, EP), jnp.float32),
          jax.ShapeDtypeStruct((B, E), jnp.float32),
      ],
  )(v2e_t, u2e_t, wv, nodes)


def _tc_body(emb_ref, euv_ref, u_ref, r_ref, r2e_ref,
             w3_ref, bc3_ref, w5_ref, bc5_ref,
             w1b_ref, w1c3_ref, w1c5_ref, b1_ref,
             w2_ref, b2_ref,
             a1o_ref, a1u_ref, a1b_ref, a2_ref, a2b_ref, a3_ref, a3b_ref,
             out_ref):
  f32 = jnp.float32
  dot = lambda a, b: lax.dot(a, b, preferred_element_type=f32)
  relu = lambda x: jnp.maximum(x, 0.0)

  emb = emb_ref[...]

  w3 = w3_ref[...]
  acc3 = dot(emb[:, 0:3 * WD], w3)
  for l in range(1, SEQ - 3 + 1):
    acc3 = jnp.maximum(acc3, dot(emb[:, l * WD:(l + 3) * WD], w3))
  e3 = relu(acc3 + bc3_ref[...])

  w5 = w5_ref[...]
  acc5 = dot(emb[:, 0:5 * WD], w5)
  for l in range(1, SEQ - 5 + 1):
    acc5 = jnp.maximum(acc5, dot(emb[:, l * WD:(l + 5) * WD], w5))
  e5 = relu(acc5 + bc5_ref[...])

  euv_p = euv_ref[...][:, :E]
  r = r_ref[...]
  onehot = (lax.broadcasted_iota(jnp.int32, (NSB, 8), 1) == r).astype(f32)
  rmat = dot(r2e_ref[...], w1b_ref[...])
  h = (euv_p + dot(onehot, rmat)
       + dot(e3, w1c3_ref[...]) + dot(e5, w1c5_ref[...]) + b1_ref[...])
  h = relu(h)
  o = relu(dot(h, w2_ref[...]) + b2_ref[...])

  ua = dot(u_ref[...], a1u_ref[...])
  ua_b = jnp.broadcast_to(ua[:, None, :], (NB, HIST, E)).reshape(NSB, E)
  a = relu(dot(o, a1o_ref[...]) + ua_b + a1b_ref[...])
  a = relu(dot(a, a2_ref[...]) + a2b_ref[...])
  s = dot(a, a3_ref[...]) + a3b_ref[...]

  srs = s.reshape(NB, HIST)
  m = jnp.max(srs, axis=1, keepdims=True)
  ex = jnp.exp(srs - m)
  wgt = ex / jnp.sum(ex, axis=1, keepdims=True)

  o3 = o.reshape(NB, HIST, E)
  out_ref[...] = jnp.sum(o3 * wgt[:, :, None], axis=1)


def _full_spec(shape):
  nd = len(shape)
  return pl.BlockSpec(shape, lambda i, _n=nd: (0,) * _n)


def kernel(nodes, history_uv, history_r, history_w,
           v2e_w, u2e_w, r2e_w, word_emb,
           Wc3, bc3, Wc5, bc5,
           W1, b1, W2, b2,
           A1, a1, A2, a2, A3, a3):
  f32 = jnp.float32
  w_idx3d = history_w.astype(jnp.int32).reshape(NW, W_NCH, W_CHUNK)
  v_idx3d = history_uv.astype(jnp.int32).reshape(V_NCH, 1, V_CHUNK)
  u_idx3d = nodes.astype(jnp.int32).reshape(U_NCH, 1, V_CHUNK)

  w3 = Wc3.reshape(NF, 3 * WD).T
  w5 = Wc5.reshape(NF, 5 * WD).T
  w1t = W1.T
  w1a = jnp.pad(w1t[:E], ((0, 0), (0, EP - E)))
  w1b = w1t[E:2 * E]
  w1c3 = w1t[2 * E:2 * E + NF]
  w1c5 = w1t[2 * E + NF:]
  r2e8 = jnp.zeros((8, E), f32).at[:6].set(r2e_w)
  a1t = A1.T
  a1o = a1t[:E]
  a1u = a1t[E:]

  w_rows = _sc_word(w_idx3d, word_emb)
  v_proj, u_rows = _proj_tables(v2e_w.T, u2e_w.T, w1a,
                                nodes.astype(jnp.int32).reshape(B, 1))
  v_rows = _sc_v(v_idx3d, v_proj)
  emb = w_rows.reshape(NSEQ, SEQ * WD)

  grid_spec = pl.GridSpec(
      grid=(GRID,),
      in_specs=[
          pl.BlockSpec((NSB, SEQ * WD), lambda i: (i, 0)),
          pl.BlockSpec((NSB, EP), lambda i: (i, 0)),
          pl.BlockSpec((NB, E), lambda i: (i, 0)),
          pl.BlockSpec((NSB, 1), lambda i: (i, 0)),
          _full_spec((8, E)),
          _full_spec((3 * WD, NF)),
          _full_spec((1, NF)),
          _full_spec((5 * WD, NF)),
          _full_spec((1, NF)),
          _full_spec((E, E)),
          _full_spec((NF, E)),
          _full_spec((NF, E)),
          _full_spec((1, E)),
          _full_spec((E, E)),
          _full_spec((1, E)),
          _full_spec((E, E)),
          _full_spec((E, E)),
          _full_spec((1, E)),
          _full_spec((E, E)),
          _full_spec((1, E)),
          _full_spec((E, 1)),
          _full_spec((1, 1)),
      ],
      out_specs=pl.BlockSpec((NB, E), lambda i: (i, 0)),
  )

  out = pl.pallas_call(
      _tc_body,
      grid_spec=grid_spec,
      out_shape=jax.ShapeDtypeStruct((B, E), f32),
  )(
      emb, v_rows, u_rows, history_r.astype(jnp.int32).reshape(NSEQ, 1), r2e8,
      w3, bc3.reshape(1, NF), w5, bc5.reshape(1, NF),
      w1b, w1c3, w1c5, b1.reshape(1, E),
      W2.T, b2.reshape(1, E),
      a1o, a1u, a1.reshape(1, E), A2.T, a2.reshape(1, E),
      A3.T, a3.reshape(1, 1),
  )
  return out
